# baseline (speedup 1.0000x reference)
import functools

import jax
import jax.numpy as jnp
from jax import lax
from jax.experimental import pallas as pl
from jax.experimental.pallas import tpu as pltpu
from jax.experimental.pallas import tpu_sc as plsc

B = 16384
NC, NS = 2, 16
NW = NC * NS
BPW = B // NW
GRP = 16
NGRP = BPW // GRP
SUB = 8
NSUB = BPW // SUB
DEPTH = 3
SVOCAB = 1000
SDIM = 16
IDIM = 32
DOUT = 96


def _emb_body(s_hbm, d_hbm, c_hbm, st_hbm, it_hbm,
              Wst, Wdt, Wct, Wstt, Wit, out_hbm,
              idx_v, lsem, osem):
    wid = lax.axis_index("s") * NC + lax.axis_index("c")
    base = wid * BPW
    idx_hbms = (s_hbm, d_hbm, c_hbm, st_hbm, it_hbm)

    loads = [pltpu.async_copy(idx_hbms[k].at[pl.ds(base, BPW)],
                              idx_v.at[pl.ds(k * BPW, BPW)], lsem)
             for k in range(5)]
    for cp in loads:
        cp.wait()

    iota = lax.iota(jnp.int32, 16)

    def small_phase(t0, t1, t2, t3, sblk, tsem):
        tabs = (t0, t1, t2, t3)
        tloads = [pltpu.async_copy(t, d, tsem)
                  for t, d in zip((Wst, Wdt, Wct, Wstt), tabs)]
        for cp in tloads:
            cp.wait()
        for t in range(4):
            for c in range(SDIM):
                c_vec = jnp.full((16,), c, jnp.int32)
                row_vec = jnp.full((16,), t * SDIM + c, jnp.int32)

                def g_body(g, carry, t=t, c_vec=c_vec, row_vec=row_vec):
                    idx16 = idx_v[pl.ds(t * BPW + g * GRP, GRP)]
                    vals = plsc.load_gather(tabs[t], [c_vec, idx16])
                    plsc.store_scatter(sblk, [row_vec, g * GRP + iota],
                                       vals)
                    return carry

                lax.fori_loop(0, NGRP, g_body, 0)
        pltpu.async_copy(
            sblk, out_hbm.at[pl.ds(0, 64), pl.ds(base, BPW)], tsem).wait()

    def item_phase(ibuf, iblk, isem):
        def fire(k):
            ids = idx_v[pl.ds(4 * BPW + k * SUB, 16)]
            sbase = lax.rem(k, DEPTH) * SUB
            for j in range(SUB):
                wstart = pl.multiple_of((ids[j] >> 7) << 7, 128)
                for cb in range(4):
                    pltpu.async_copy(
                        Wit.at[pl.ds(cb * 8, 8), pl.ds(wstart, 128)],
                        ibuf.at[pl.ds((sbase + j) * IDIM + cb * 8, 8)], isem)

        def drain_extract(k):
            ids = idx_v[pl.ds(4 * BPW + k * SUB, 16)]
            lanes = ids & 127
            sbase = lax.rem(k, DEPTH) * SUB
            for j in range(SUB):
                pltpu.make_async_copy(
                    Wit.at[:, pl.ds(0, 128)],
                    ibuf.at[pl.ds((sbase + j) * IDIM, IDIM)], isem).wait()
                lane_vec = jnp.full((16,), lanes[j], jnp.int32)
                col_vec = k * SUB + j + jnp.zeros((16,), jnp.int32)
                buf = ibuf.at[pl.ds((sbase + j) * IDIM, IDIM)]
                lo = plsc.load_gather(buf, [iota, lane_vec])
                hi = plsc.load_gather(buf, [iota + 16, lane_vec])
                plsc.store_scatter(iblk, [iota, col_vec], lo)
                plsc.store_scatter(iblk, [16 + iota, col_vec], hi)

        for k in range(DEPTH):
            fire(k)

        def k_body(k, carry):
            drain_extract(k - DEPTH)
            fire(k)
            return carry

        lax.fori_loop(DEPTH, NSUB, k_body, 0)
        for k in range(NSUB - DEPTH, NSUB):
            drain_extract(k)

        pltpu.async_copy(
            iblk, out_hbm.at[pl.ds(64, 32), pl.ds(base, BPW)], isem).wait()

    pl.run_scoped(small_phase,
                  pltpu.VMEM((SDIM, SVOCAB), jnp.float32),
                  pltpu.VMEM((SDIM, SVOCAB), jnp.float32),
                  pltpu.VMEM((SDIM, SVOCAB), jnp.float32),
                  pltpu.VMEM((SDIM, SVOCAB), jnp.float32),
                  pltpu.VMEM((64, BPW), jnp.float32),
                  pltpu.SemaphoreType.DMA)
    pl.run_scoped(item_phase,
                  pltpu.VMEM((DEPTH * SUB * IDIM, 128), jnp.float32),
                  pltpu.VMEM((IDIM, BPW), jnp.float32),
                  pltpu.SemaphoreType.DMA)


def kernel(store_id, dept_id, cat_id, state_id, item_id,
           W_store_id, W_dept_id, W_cat_id, W_state_id, W_item_id):
    mesh = plsc.VectorSubcoreMesh(core_axis_name="c", subcore_axis_name="s",
                                  num_cores=NC, num_subcores=NS)
    run = pl.kernel(
        _emb_body,
        out_type=jax.ShapeDtypeStruct((DOUT, B), jnp.float32),
        mesh=mesh,
        compiler_params=pltpu.CompilerParams(needs_layout_passes=False),
        scratch_types=[
            pltpu.VMEM((5 * BPW + 16,), jnp.int32),
            pltpu.SemaphoreType.DMA,
            pltpu.SemaphoreType.DMA,
        ],
    )
    out_t = run(store_id, dept_id, cat_id, state_id, item_id,
                W_store_id.T, W_dept_id.T, W_cat_id.T, W_state_id.T,
                W_item_id.T)
    return out_t.T


# speedup vs baseline: 1.0226x; 1.0226x over previous
import functools

import jax
import jax.numpy as jnp
from jax import lax
from jax.experimental import pallas as pl
from jax.experimental.pallas import tpu as pltpu
from jax.experimental.pallas import tpu_sc as plsc

B = 16384
NC, NS = 2, 16
NW = NC * NS
BPW = B // NW
GRP = 16
NGRP = BPW // GRP
SUB = 8
NSUB = BPW // SUB
DEPTH = 3
SVOCAB = 1000
SDIM = 16
IDIM = 32
DOUT = 96


def _emb_body(s_hbm, d_hbm, c_hbm, st_hbm, it_hbm,
              Wst, Wdt, Wct, Wstt, Wit, out_hbm,
              idx_v, lsem, osem):
    wid = lax.axis_index("s") * NC + lax.axis_index("c")
    base = wid * BPW
    idx_hbms = (s_hbm, d_hbm, c_hbm, st_hbm, it_hbm)

    loads = [pltpu.async_copy(idx_hbms[k].at[pl.ds(base, BPW)],
                              idx_v.at[pl.ds(k * BPW, BPW)], lsem)
             for k in range(5)]
    for cp in loads:
        cp.wait()

    iota = lax.iota(jnp.int32, 16)

    def small_phase(t0, t1, t2, t3, sblk, tsem):
        tabs = (t0, t1, t2, t3)
        tloads = [pltpu.async_copy(t, d, tsem)
                  for t, d in zip((Wst, Wdt, Wct, Wstt), tabs)]
        for cp in tloads:
            cp.wait()
        for t in range(4):
            for c in range(SDIM):
                c_vec = jnp.full((16,), c, jnp.int32)
                row_vec = jnp.full((16,), t * SDIM + c, jnp.int32)

                def g_body(g, carry, t=t, c_vec=c_vec, row_vec=row_vec):
                    idx16 = idx_v[pl.ds(t * BPW + g * GRP, GRP)]
                    vals = plsc.load_gather(tabs[t], [c_vec, idx16])
                    plsc.store_scatter(sblk, [row_vec, g * GRP + iota],
                                       vals)
                    return carry

                lax.fori_loop(0, NGRP, g_body, 0)
        pltpu.async_copy(
            sblk, out_hbm.at[pl.ds(0, 64), pl.ds(base, BPW)], tsem).wait()

    def item_phase(ibuf, iblk, isem):
        def fire(k):
            ids = idx_v[pl.ds(4 * BPW + k * SUB, 16)]
            sbase = lax.rem(k, DEPTH) * SUB
            for j in range(SUB):
                wstart = pl.multiple_of((ids[j] >> 7) << 7, 128)
                pltpu.async_copy(
                    Wit.at[:, pl.ds(wstart, 128)],
                    ibuf.at[pl.ds((sbase + j) * IDIM, IDIM)], isem)

        def drain_extract(k):
            ids = idx_v[pl.ds(4 * BPW + k * SUB, 16)]
            lanes = ids & 127
            sbase = lax.rem(k, DEPTH) * SUB
            pltpu.make_async_copy(
                Wit.at[pl.ds(0, SUB * IDIM), pl.ds(0, 128)],
                ibuf.at[pl.ds(sbase * IDIM, SUB * IDIM)], isem).wait()
            for j in range(SUB):
                lane_vec = jnp.full((16,), lanes[j], jnp.int32)
                col_vec = k * SUB + j + jnp.zeros((16,), jnp.int32)
                buf = ibuf.at[pl.ds((sbase + j) * IDIM, IDIM)]
                lo = plsc.load_gather(buf, [iota, lane_vec])
                hi = plsc.load_gather(buf, [iota + 16, lane_vec])
                plsc.store_scatter(iblk, [iota, col_vec], lo)
                plsc.store_scatter(iblk, [16 + iota, col_vec], hi)

        for k in range(DEPTH):
            fire(k)

        def k_body(k, carry):
            drain_extract(k - DEPTH)
            fire(k)
            return carry

        lax.fori_loop(DEPTH, NSUB, k_body, 0)
        for k in range(NSUB - DEPTH, NSUB):
            drain_extract(k)

        pltpu.async_copy(
            iblk, out_hbm.at[pl.ds(64, 32), pl.ds(base, BPW)], isem).wait()

    pl.run_scoped(small_phase,
                  pltpu.VMEM((SDIM, SVOCAB), jnp.float32),
                  pltpu.VMEM((SDIM, SVOCAB), jnp.float32),
                  pltpu.VMEM((SDIM, SVOCAB), jnp.float32),
                  pltpu.VMEM((SDIM, SVOCAB), jnp.float32),
                  pltpu.VMEM((64, BPW), jnp.float32),
                  pltpu.SemaphoreType.DMA)
    pl.run_scoped(item_phase,
                  pltpu.VMEM((DEPTH * SUB * IDIM, 128), jnp.float32),
                  pltpu.VMEM((IDIM, BPW), jnp.float32),
                  pltpu.SemaphoreType.DMA)


def kernel(store_id, dept_id, cat_id, state_id, item_id,
           W_store_id, W_dept_id, W_cat_id, W_state_id, W_item_id):
    mesh = plsc.VectorSubcoreMesh(core_axis_name="c", subcore_axis_name="s",
                                  num_cores=NC, num_subcores=NS)
    run = pl.kernel(
        _emb_body,
        out_type=jax.ShapeDtypeStruct((DOUT, B), jnp.float32),
        mesh=mesh,
        compiler_params=pltpu.CompilerParams(needs_layout_passes=False),
        scratch_types=[
            pltpu.VMEM((5 * BPW + 16,), jnp.int32),
            pltpu.SemaphoreType.DMA,
            pltpu.SemaphoreType.DMA,
        ],
    )
    out_t = run(store_id, dept_id, cat_id, state_id, item_id,
                W_store_id.T, W_dept_id.T, W_cat_id.T, W_state_id.T,
                W_item_id.T)
    return out_t.T
